# baseline (speedup 1.0000x reference)
import functools

import jax
import jax.numpy as jnp
from jax import lax
from jax.experimental import pallas as pl
from jax.experimental.pallas import tpu as pltpu
from jax.experimental.pallas import tpu_sc as plsc


def _sc_gather(tables, idx_fm, N, D):
    info = plsc.get_sparse_core_info()
    NC, NS = info.num_cores, info.num_subcores
    NW = NC * NS
    rows_w = N // NW
    mesh = plsc.VectorSubcoreMesh(core_axis_name="c", subcore_axis_name="s")

    @functools.partial(
        pl.kernel,
        mesh=mesh,
        compiler_params=pltpu.CompilerParams(use_tc_tiling_on_sc=False),
        out_type=jax.ShapeDtypeStruct((4 * N, D), jnp.float32),
        scratch_types=[
            pltpu.VMEM((4, rows_w), jnp.int32),
            pltpu.VMEM((2, rows_w, D), jnp.float32),
            pltpu.SemaphoreType.DMA,
            pltpu.SemaphoreType.DMA,
        ],
    )
    def k(t0, t1, t2, t3, idx_hbm, out_hbm, idx_v, rows_v, gsem, ssem):
        wid = lax.axis_index("s") * NC + lax.axis_index("c")
        base = wid * rows_w
        tabs = (t0, t1, t2, t3)
        for f in range(4):
            pltpu.sync_copy(idx_hbm.at[pl.ds(f * N + base, rows_w)], idx_v.at[f])
        gathers = []
        for f in range(4):
            gathers.append(
                pltpu.async_copy(tabs[f].at[idx_v.at[f]], rows_v.at[f % 2], gsem)
            )
            if f >= 1:
                gathers[f - 1].wait()
                pltpu.async_copy(
                    rows_v.at[(f - 1) % 2],
                    out_hbm.at[pl.ds((f - 1) * N + base, rows_w)],
                    ssem,
                ).wait()
        gathers[3].wait()
        pltpu.async_copy(
            rows_v.at[1], out_hbm.at[pl.ds(3 * N + base, rows_w)], ssem
        ).wait()

    return k(*tables, idx_fm)


def _mlp_body(x0_ref, x1_ref, x2_ref, x3_ref, w1_ref, b1_ref, w2_ref, b2_ref, o_ref):
    x = jnp.concatenate(
        [x0_ref[...], x1_ref[...], x2_ref[...], x3_ref[...]], axis=1
    )
    h = jnp.dot(x, w1_ref[...], preferred_element_type=jnp.float32)
    h = jnp.maximum(h + b1_ref[...], 0.0)
    o_ref[...] = (
        jnp.dot(h, w2_ref[...], preferred_element_type=jnp.float32) + b2_ref[...]
    )


def _tc_mlp(emb_fm, N, E, W1, b1, W2, b2):
    H = W1.shape[1]
    O = W2.shape[1]
    BN = 1024
    nb = N // BN
    x_specs = [
        pl.BlockSpec((BN, E), functools.partial(lambda i, f: (f * nb + i, 0), f=f))
        for f in range(4)
    ]
    return pl.pallas_call(
        _mlp_body,
        grid=(nb,),
        in_specs=x_specs
        + [
            pl.BlockSpec((4 * E, H), lambda i: (0, 0)),
            pl.BlockSpec((1, H), lambda i: (0, 0)),
            pl.BlockSpec((H, O), lambda i: (0, 0)),
            pl.BlockSpec((1, O), lambda i: (0, 0)),
        ],
        out_specs=pl.BlockSpec((BN, O), lambda i: (i, 0)),
        out_shape=jax.ShapeDtypeStruct((N, O), jnp.float32),
    )(emb_fm, emb_fm, emb_fm, emb_fm, W1, b1.reshape(1, H), W2, b2.reshape(1, O))


def kernel(input_tensor, genre_table, key_table, meter_table, unl_table, W1, b1, W2, b2):
    N = input_tensor.shape[0]
    V, E = genre_table.shape
    idx_fm = input_tensor.T.reshape(-1)
    emb_fm = _sc_gather(
        (genre_table, key_table, meter_table, unl_table), idx_fm, N, E
    )
    return _tc_mlp(emb_fm, N, E, W1, b1, W2, b2)


# speedup vs baseline: 1.3066x; 1.3066x over previous
import functools

import jax
import jax.numpy as jnp
from jax import lax
from jax.experimental import pallas as pl
from jax.experimental.pallas import tpu as pltpu
from jax.experimental.pallas import tpu_sc as plsc


def _sc_gather(tables, idx_fm, N, D):
    info = plsc.get_sparse_core_info()
    NC, NS = info.num_cores, info.num_subcores
    NW = NC * NS
    W_PER_F = NW // 4
    rows_w = N // W_PER_F
    CH = rows_w // 2
    mesh = plsc.VectorSubcoreMesh(core_axis_name="c", subcore_axis_name="s")

    @functools.partial(
        pl.kernel,
        mesh=mesh,
        compiler_params=pltpu.CompilerParams(use_tc_tiling_on_sc=False),
        out_type=jax.ShapeDtypeStruct((4 * N, D), jnp.float32),
        scratch_types=[
            pltpu.VMEM((rows_w,), jnp.int32),
            pltpu.VMEM((CH, D), jnp.float32),
            pltpu.SemaphoreType.DMA,
        ],
    )
    def k(t0, t1, t2, t3, idx_hbm, out_hbm, idx_v, rows_v, gsem):
        wid = lax.axis_index("s") * NC + lax.axis_index("c")
        f = wid // W_PER_F
        base = (wid % W_PER_F) * rows_w
        tabs = (t0, t1, t2, t3)
        for ff in range(4):
            @pl.when(f == ff)
            def _():
                pltpu.sync_copy(
                    idx_hbm.at[pl.ds(ff * N + base, rows_w)], idx_v
                )
                for c in range(2):
                    pltpu.async_copy(
                        tabs[ff].at[idx_v.at[pl.ds(c * CH, CH)]], rows_v, gsem
                    ).wait()
                    pltpu.sync_copy(
                        rows_v,
                        out_hbm.at[pl.ds(ff * N + base + c * CH, CH)],
                    )

    return k(*tables, idx_fm)


def _refold(x, BN, E):
    z = jnp.broadcast_to(x[:, None, :], (BN // 2, 2, 2 * E)).reshape(BN, 2 * E)
    even = (lax.broadcasted_iota(jnp.int32, (BN, E), 0) % 2) == 0
    return jnp.where(even, z[:, :E], z[:, E:])


def _mlp_body(x0_ref, x1_ref, x2_ref, x3_ref, w1_ref, b1_ref, w2_ref, b2_ref, o_ref, *, BN, E):
    x = jnp.concatenate(
        [_refold(xr[...], BN, E) for xr in (x0_ref, x1_ref, x2_ref, x3_ref)],
        axis=1,
    )
    h = jnp.dot(x, w1_ref[...], preferred_element_type=jnp.float32)
    h = jnp.maximum(h + b1_ref[...], 0.0)
    o_ref[...] = (
        jnp.dot(h, w2_ref[...], preferred_element_type=jnp.float32) + b2_ref[...]
    )


def _tc_mlp(emb2, N, E, W1, b1, W2, b2):
    H = W1.shape[1]
    O = W2.shape[1]
    BN = 1024
    nb = N // BN
    half_blocks = (N // 2) // (BN // 2)
    x_specs = [
        pl.BlockSpec(
            (BN // 2, 2 * E),
            functools.partial(lambda i, f: (f * half_blocks + i, 0), f=f),
        )
        for f in range(4)
    ]
    return pl.pallas_call(
        functools.partial(_mlp_body, BN=BN, E=E),
        grid=(nb,),
        in_specs=x_specs
        + [
            pl.BlockSpec((4 * E, H), lambda i: (0, 0)),
            pl.BlockSpec((1, H), lambda i: (0, 0)),
            pl.BlockSpec((H, O), lambda i: (0, 0)),
            pl.BlockSpec((1, O), lambda i: (0, 0)),
        ],
        out_specs=pl.BlockSpec((BN, O), lambda i: (i, 0)),
        out_shape=jax.ShapeDtypeStruct((N, O), jnp.float32),
    )(emb2, emb2, emb2, emb2, W1, b1.reshape(1, H), W2, b2.reshape(1, O))


def kernel(input_tensor, genre_table, key_table, meter_table, unl_table, W1, b1, W2, b2):
    N = input_tensor.shape[0]
    V, E = genre_table.shape
    idx_fm = input_tensor.T.reshape(-1)
    emb = _sc_gather(
        (genre_table, key_table, meter_table, unl_table), idx_fm, N, E
    )
    emb2 = emb.reshape(2 * N, 2 * E)
    return _tc_mlp(emb2, N, E, W1, b1, W2, b2)
